# baseline (speedup 1.0000x reference)
import functools

import jax
import jax.numpy as jnp
from jax import lax
from jax.experimental import pallas as pl
from jax.experimental.pallas import tpu as pltpu
from jax.experimental.pallas import tpu_sc as plsc

B = 16384
D = 64
VOC = 100000
NC = 2
NS = 16
RPW = 8
QB = 8192


@functools.cache
def _make_sc_gather():
    mesh = plsc.VectorSubcoreMesh(core_axis_name="c", subcore_axis_name="s")

    @functools.partial(
        pl.kernel,
        out_type=[jax.ShapeDtypeStruct((D, B), jnp.float32) for _ in range(4)],
        mesh=mesh,
        scratch_types=[
            pltpu.VMEM((B,), jnp.int32),
            pltpu.VMEM((VOC,), jnp.float32),
            pltpu.VMEM((QB,), jnp.float32),
            pltpu.SemaphoreType.DMA,
        ],
        compiler_params=pltpu.CompilerParams(needs_layout_passes=False),
    )
    def _sc_gather(umlpT, imlpT, umfT, imfT, uidx, iidx,
                   o_umlp, o_imlp, o_umf, o_imf,
                   idx_v, rowbuf, outq, sem):
        c = lax.axis_index("c")
        s = lax.axis_index("s")
        tbl = s // 4
        g = (s % 4) * NC + c

        def make_branch(table, idx_hbm, out):
            def br():
                pltpu.sync_copy(idx_hbm, idx_v)
                for r in range(RPW):
                    d = g * RPW + r
                    pltpu.sync_copy(table.at[d], rowbuf)
                    for q in range(B // QB):
                        @plsc.parallel_loop(0, QB, step=16, unroll=8)
                        def grp(j):
                            iv = idx_v[pl.ds(q * QB + j, 16)]
                            outq[pl.ds(j, 16)] = plsc.load_gather(rowbuf, [iv])

                        pltpu.sync_copy(outq, out.at[d, pl.ds(q * QB, QB)])
            return br

        lax.switch(tbl, [
            make_branch(umlpT, uidx, o_umlp),
            make_branch(imlpT, iidx, o_imlp),
            make_branch(umfT, uidx, o_umf),
            make_branch(imfT, iidx, o_imf),
        ])

    return _sc_gather


def _tc_body(u_ref, i_ref, uf_ref, if_ref,
             w0u_ref, w0i_ref, b0_ref, w1_ref, b1_ref, w2_ref, b2_ref,
             wamlp_ref, wamf_ref, ba_ref, out_ref):
    h = jnp.dot(w0u_ref[...], u_ref[...], preferred_element_type=jnp.float32)
    h += jnp.dot(w0i_ref[...], i_ref[...], preferred_element_type=jnp.float32)
    h = jnp.maximum(h + b0_ref[...], 0.0)
    h = jnp.maximum(
        jnp.dot(w1_ref[...], h, preferred_element_type=jnp.float32) + b1_ref[...], 0.0)
    h = jnp.maximum(
        jnp.dot(w2_ref[...], h, preferred_element_type=jnp.float32) + b2_ref[...], 0.0)
    mf = uf_ref[...] * if_ref[...]
    logit = (jnp.dot(wamlp_ref[...], h, preferred_element_type=jnp.float32)
             + jnp.dot(wamf_ref[...], mf, preferred_element_type=jnp.float32)
             + ba_ref[...])
    out_ref[...] = jax.nn.sigmoid(logit)


def kernel(user_indices, item_indices, user_mlp, item_mlp, user_mf, item_mf,
           W0, b0, W1, b1, W2, b2, Wa, ba):
    uidx = user_indices.astype(jnp.int32)
    iidx = item_indices.astype(jnp.int32)
    u_T, i_T, uf_T, if_T = _make_sc_gather()(
        user_mlp.T, item_mlp.T, user_mf.T, item_mf.T, uidx, iidx)

    w0u = W0[:, :D]
    w0i = W0[:, D:]
    wamlp = Wa[:, :32]
    wamf = Wa[:, 32:]
    b0r = b0.reshape(-1, 1)
    b1r = b1.reshape(-1, 1)
    b2r = b2.reshape(-1, 1)
    bar = ba.reshape(1, 1)

    BT = 2048
    nblk = B // BT
    row_spec = pl.BlockSpec((D, BT), lambda b: (0, b))
    full = lambda shape: pl.BlockSpec(shape, lambda b: tuple(0 for _ in shape))
    out = pl.pallas_call(
        _tc_body,
        grid=(nblk,),
        in_specs=[
            row_spec, row_spec, row_spec, row_spec,
            full((128, D)), full((128, D)), full((128, 1)),
            full((D, 128)), full((D, 1)),
            full((32, D)), full((32, 1)),
            full((1, 32)), full((1, D)), full((1, 1)),
        ],
        out_specs=pl.BlockSpec((1, BT), lambda b: (0, b)),
        out_shape=jax.ShapeDtypeStruct((1, B), jnp.float32),
    )(u_T, i_T, uf_T, if_T,
      w0u, w0i, b0r, W1, b1r, W2, b2r, wamlp, wamf, bar)
    return out.reshape(B, 1)


# speedup vs baseline: 1.0055x; 1.0055x over previous
import functools

import jax
import jax.numpy as jnp
from jax import lax
from jax.experimental import pallas as pl
from jax.experimental.pallas import tpu as pltpu
from jax.experimental.pallas import tpu_sc as plsc

B = 16384
D = 64
VOC = 100000
NC = 2
NS = 16
RPW = 8
QB = 4096


@functools.cache
def _make_sc_gather():
    mesh = plsc.VectorSubcoreMesh(core_axis_name="c", subcore_axis_name="s")

    @functools.partial(
        pl.kernel,
        out_type=[jax.ShapeDtypeStruct((D, B), jnp.float32) for _ in range(4)],
        mesh=mesh,
        scratch_types=[
            pltpu.VMEM((B,), jnp.int32),
            pltpu.VMEM((VOC,), jnp.float32),
            pltpu.VMEM((QB,), jnp.float32),
            pltpu.VMEM((QB,), jnp.float32),
            pltpu.SemaphoreType.DMA,
            pltpu.SemaphoreType.DMA,
        ],
        compiler_params=pltpu.CompilerParams(needs_layout_passes=False),
    )
    def _sc_gather(umlpT, imlpT, umfT, imfT, uidx, iidx,
                   o_umlp, o_imlp, o_umf, o_imf,
                   idx_v, rowbuf, outq0, outq1, sem, osem):
        c = lax.axis_index("c")
        s = lax.axis_index("s")
        tbl = s // 4
        g = (s % 4) * NC + c

        def make_branch(table, idx_hbm, out):
            def br():
                pltpu.sync_copy(idx_hbm, idx_v)
                bufs = (outq0, outq1)
                pend = [None, None]
                for r in range(RPW):
                    d = g * RPW + r
                    pltpu.sync_copy(table.at[d], rowbuf)
                    for q in range(B // QB):
                        p = q % 2
                        buf = bufs[p]
                        if pend[p] is not None:
                            pend[p].wait()

                        @plsc.parallel_loop(0, QB, step=16, unroll=8)
                        def grp(j):
                            iv = idx_v[pl.ds(q * QB + j, 16)]
                            buf[pl.ds(j, 16)] = plsc.load_gather(rowbuf, [iv])

                        pend[p] = pltpu.async_copy(
                            buf, out.at[d, pl.ds(q * QB, QB)], osem)
                for p in range(2):
                    if pend[p] is not None:
                        pend[p].wait()
            return br

        lax.switch(tbl, [
            make_branch(umlpT, uidx, o_umlp),
            make_branch(imlpT, iidx, o_imlp),
            make_branch(umfT, uidx, o_umf),
            make_branch(imfT, iidx, o_imf),
        ])

    return _sc_gather


def _tc_body(u_ref, i_ref, uf_ref, if_ref,
             w0u_ref, w0i_ref, b0_ref, w1_ref, b1_ref, w2_ref, b2_ref,
             wamlp_ref, wamf_ref, ba_ref, out_ref):
    h = jnp.dot(w0u_ref[...], u_ref[...], preferred_element_type=jnp.float32)
    h += jnp.dot(w0i_ref[...], i_ref[...], preferred_element_type=jnp.float32)
    h = jnp.maximum(h + b0_ref[...], 0.0)
    h = jnp.maximum(
        jnp.dot(w1_ref[...], h, preferred_element_type=jnp.float32) + b1_ref[...], 0.0)
    h = jnp.maximum(
        jnp.dot(w2_ref[...], h, preferred_element_type=jnp.float32) + b2_ref[...], 0.0)
    mf = uf_ref[...] * if_ref[...]
    logit = (jnp.dot(wamlp_ref[...], h, preferred_element_type=jnp.float32)
             + jnp.dot(wamf_ref[...], mf, preferred_element_type=jnp.float32)
             + ba_ref[...])
    out_ref[...] = jax.nn.sigmoid(logit)


def kernel(user_indices, item_indices, user_mlp, item_mlp, user_mf, item_mf,
           W0, b0, W1, b1, W2, b2, Wa, ba):
    uidx = user_indices.astype(jnp.int32)
    iidx = item_indices.astype(jnp.int32)
    u_T, i_T, uf_T, if_T = _make_sc_gather()(
        user_mlp.T, item_mlp.T, user_mf.T, item_mf.T, uidx, iidx)

    w0u = W0[:, :D]
    w0i = W0[:, D:]
    wamlp = Wa[:, :32]
    wamf = Wa[:, 32:]
    b0r = b0.reshape(-1, 1)
    b1r = b1.reshape(-1, 1)
    b2r = b2.reshape(-1, 1)
    bar = ba.reshape(1, 1)

    BT = 2048
    nblk = B // BT
    row_spec = pl.BlockSpec((D, BT), lambda b: (0, b))
    full = lambda shape: pl.BlockSpec(shape, lambda b: tuple(0 for _ in shape))
    out = pl.pallas_call(
        _tc_body,
        grid=(nblk,),
        in_specs=[
            row_spec, row_spec, row_spec, row_spec,
            full((128, D)), full((128, D)), full((128, 1)),
            full((D, 128)), full((D, 1)),
            full((32, D)), full((32, 1)),
            full((1, 32)), full((1, D)), full((1, 1)),
        ],
        out_specs=pl.BlockSpec((1, BT), lambda b: (0, b)),
        out_shape=jax.ShapeDtypeStruct((1, B), jnp.float32),
    )(u_T, i_T, uf_T, if_T,
      w0u, w0i, b0r, W1, b1r, W2, b2r, wamlp, wamf, bar)
    return out.reshape(B, 1)


# speedup vs baseline: 1.0304x; 1.0248x over previous
import functools

import jax
import jax.numpy as jnp
from jax import lax
from jax.experimental import pallas as pl
from jax.experimental.pallas import tpu as pltpu
from jax.experimental.pallas import tpu_sc as plsc

B = 16384
D = 64
VOC = 100000
NC = 2
NS = 16
RPW = 8
QB = 4096


@functools.cache
def _make_sc_gather():
    mesh = plsc.VectorSubcoreMesh(core_axis_name="c", subcore_axis_name="s")

    @functools.partial(
        pl.kernel,
        out_type=[jax.ShapeDtypeStruct((D, B), jnp.float32) for _ in range(4)],
        mesh=mesh,
        scratch_types=[
            pltpu.VMEM((B,), jnp.int32),
            pltpu.VMEM((VOC,), jnp.float32),
            pltpu.VMEM((QB,), jnp.float32),
            pltpu.VMEM((QB,), jnp.float32),
            pltpu.SemaphoreType.DMA,
            pltpu.SemaphoreType.DMA,
        ],
        compiler_params=pltpu.CompilerParams(needs_layout_passes=False),
    )
    def _sc_gather(umlpT, imlpT, umfT, imfT, uidx, iidx,
                   o_umlp, o_imlp, o_umf, o_imf,
                   idx_v, rowbuf, outq0, outq1, sem, osem):
        c = lax.axis_index("c")
        s = lax.axis_index("s")
        tbl = s // 4
        g = (s % 4) * NC + c

        def make_branch(table, idx_hbm, out):
            def br():
                pltpu.sync_copy(idx_hbm, idx_v)
                bufs = (outq0, outq1)
                pend = [None, None]
                for r in range(RPW):
                    d = g * RPW + r
                    pltpu.sync_copy(table.at[d], rowbuf)
                    for q in range(B // QB):
                        p = q % 2
                        buf = bufs[p]
                        if pend[p] is not None:
                            pend[p].wait()

                        @plsc.parallel_loop(0, QB, step=16, unroll=8)
                        def grp(j):
                            iv = idx_v[pl.ds(q * QB + j, 16)]
                            buf[pl.ds(j, 16)] = plsc.load_gather(rowbuf, [iv])

                        pend[p] = pltpu.async_copy(
                            buf, out.at[d, pl.ds(q * QB, QB)], osem)
                for p in range(2):
                    if pend[p] is not None:
                        pend[p].wait()
            return br

        lax.switch(tbl, [
            make_branch(umlpT, uidx, o_umlp),
            make_branch(imlpT, iidx, o_imlp),
            make_branch(umfT, uidx, o_umf),
            make_branch(imfT, iidx, o_imf),
        ])

    return _sc_gather


def _tc_body(u_ref, i_ref, uf_ref, if_ref,
             w0u_ref, w0i_ref, b0_ref, w1_ref, b1_ref, w2_ref, b2_ref,
             wamlp_ref, wamf_ref, ba_ref, out_ref):
    h = jnp.dot(w0u_ref[...], u_ref[...], preferred_element_type=jnp.float32)
    h += jnp.dot(w0i_ref[...], i_ref[...], preferred_element_type=jnp.float32)
    h = jnp.maximum(h + b0_ref[...], 0.0)
    h = jnp.maximum(
        jnp.dot(w1_ref[...], h, preferred_element_type=jnp.float32) + b1_ref[...], 0.0)
    h = jnp.maximum(
        jnp.dot(w2_ref[...], h, preferred_element_type=jnp.float32) + b2_ref[...], 0.0)
    mf = uf_ref[...] * if_ref[...]
    logit = (jnp.dot(wamlp_ref[...], h, preferred_element_type=jnp.float32)
             + jnp.dot(wamf_ref[...], mf, preferred_element_type=jnp.float32)
             + ba_ref[...])
    out_ref[...] = jax.nn.sigmoid(logit)


def kernel(user_indices, item_indices, user_mlp, item_mlp, user_mf, item_mf,
           W0, b0, W1, b1, W2, b2, Wa, ba):
    uidx = user_indices.astype(jnp.int32)
    iidx = item_indices.astype(jnp.int32)
    u_T, i_T, uf_T, if_T = _make_sc_gather()(
        user_mlp.T, item_mlp.T, user_mf.T, item_mf.T, uidx, iidx)

    w0u = W0[:, :D]
    w0i = W0[:, D:]
    wamlp = Wa[:, :32]
    wamf = Wa[:, 32:]
    b0r = b0.reshape(-1, 1)
    b1r = b1.reshape(-1, 1)
    b2r = b2.reshape(-1, 1)
    bar = ba.reshape(1, 1)

    BT = 4096
    nblk = B // BT
    row_spec = pl.BlockSpec((D, BT), lambda b: (0, b))
    full = lambda shape: pl.BlockSpec(shape, lambda b: tuple(0 for _ in shape))
    out = pl.pallas_call(
        _tc_body,
        grid=(nblk,),
        in_specs=[
            row_spec, row_spec, row_spec, row_spec,
            full((128, D)), full((128, D)), full((128, 1)),
            full((D, 128)), full((D, 1)),
            full((32, D)), full((32, 1)),
            full((1, 32)), full((1, D)), full((1, 1)),
        ],
        out_specs=pl.BlockSpec((1, BT), lambda b: (0, b)),
        out_shape=jax.ShapeDtypeStruct((1, B), jnp.float32),
    )(u_T, i_T, uf_T, if_T,
      w0u, w0i, b0r, W1, b1r, W2, b2r, wamlp, wamf, bar)
    return out.reshape(B, 1)


# speedup vs baseline: 1.0416x; 1.0108x over previous
import functools

import jax
import jax.numpy as jnp
from jax import lax
from jax.experimental import pallas as pl
from jax.experimental.pallas import tpu as pltpu
from jax.experimental.pallas import tpu_sc as plsc

B = 16384
D = 64
VOC = 100000
NC = 2
NS = 16
RPW = 8
QB = 4096


@functools.cache
def _make_sc_gather():
    mesh = plsc.VectorSubcoreMesh(core_axis_name="c", subcore_axis_name="s")

    @functools.partial(
        pl.kernel,
        out_type=[jax.ShapeDtypeStruct((D, B), jnp.float32) for _ in range(4)],
        mesh=mesh,
        scratch_types=[
            pltpu.VMEM((B,), jnp.int32),
            pltpu.VMEM((VOC,), jnp.float32),
            pltpu.VMEM((QB,), jnp.float32),
            pltpu.VMEM((QB,), jnp.float32),
            pltpu.SemaphoreType.DMA,
            pltpu.SemaphoreType.DMA,
            pltpu.SemaphoreType.DMA,
        ],
        compiler_params=pltpu.CompilerParams(needs_layout_passes=False),
    )
    def _sc_gather(umlpT, imlpT, umfT, imfT, uidx, iidx,
                   o_umlp, o_imlp, o_umf, o_imf,
                   idx_v, rowbuf, outq0, outq1, sem, osem0, osem1):
        c = lax.axis_index("c")
        s = lax.axis_index("s")
        tbl = s // 4
        g = (s % 4) * NC + c

        def make_branch(table, idx_hbm, out):
            def br():
                pltpu.sync_copy(idx_hbm, idx_v)
                bufs = (outq0, outq1)
                osems = (osem0, osem1)
                pend = [None, None]
                for r in range(RPW):
                    d = g * RPW + r
                    pltpu.sync_copy(table.at[d], rowbuf)
                    for q in range(B // QB):
                        p = q % 2
                        buf = bufs[p]
                        if pend[p] is not None:
                            pend[p].wait()

                        @plsc.parallel_loop(0, QB, step=16, unroll=8)
                        def grp(j):
                            iv = idx_v[pl.ds(q * QB + j, 16)]
                            buf[pl.ds(j, 16)] = plsc.load_gather(rowbuf, [iv])

                        pend[p] = pltpu.async_copy(
                            buf, out.at[d, pl.ds(q * QB, QB)], osems[p])
                for p in range(2):
                    if pend[p] is not None:
                        pend[p].wait()
            return br

        lax.switch(tbl, [
            make_branch(umlpT, uidx, o_umlp),
            make_branch(imlpT, iidx, o_imlp),
            make_branch(umfT, uidx, o_umf),
            make_branch(imfT, iidx, o_imf),
        ])

    return _sc_gather


def _tc_body(u_ref, i_ref, uf_ref, if_ref,
             w0u_ref, w0i_ref, b0_ref, w1_ref, b1_ref, w2_ref, b2_ref,
             wamlp_ref, wamf_ref, ba_ref, out_ref):
    h = jnp.dot(w0u_ref[...], u_ref[...], preferred_element_type=jnp.float32)
    h += jnp.dot(w0i_ref[...], i_ref[...], preferred_element_type=jnp.float32)
    h = jnp.maximum(h + b0_ref[...], 0.0)
    h = jnp.maximum(
        jnp.dot(w1_ref[...], h, preferred_element_type=jnp.float32) + b1_ref[...], 0.0)
    h = jnp.maximum(
        jnp.dot(w2_ref[...], h, preferred_element_type=jnp.float32) + b2_ref[...], 0.0)
    mf = uf_ref[...] * if_ref[...]
    logit = (jnp.dot(wamlp_ref[...], h, preferred_element_type=jnp.float32)
             + jnp.dot(wamf_ref[...], mf, preferred_element_type=jnp.float32)
             + ba_ref[...])
    out_ref[...] = jax.nn.sigmoid(logit)


def kernel(user_indices, item_indices, user_mlp, item_mlp, user_mf, item_mf,
           W0, b0, W1, b1, W2, b2, Wa, ba):
    uidx = user_indices.astype(jnp.int32)
    iidx = item_indices.astype(jnp.int32)
    u_T, i_T, uf_T, if_T = _make_sc_gather()(
        user_mlp.T, item_mlp.T, user_mf.T, item_mf.T, uidx, iidx)

    w0u = W0[:, :D]
    w0i = W0[:, D:]
    wamlp = Wa[:, :32]
    wamf = Wa[:, 32:]
    b0r = b0.reshape(-1, 1)
    b1r = b1.reshape(-1, 1)
    b2r = b2.reshape(-1, 1)
    bar = ba.reshape(1, 1)

    BT = 4096
    nblk = B // BT
    row_spec = pl.BlockSpec((D, BT), lambda b: (0, b))
    full = lambda shape: pl.BlockSpec(shape, lambda b: tuple(0 for _ in shape))
    out = pl.pallas_call(
        _tc_body,
        grid=(nblk,),
        in_specs=[
            row_spec, row_spec, row_spec, row_spec,
            full((128, D)), full((128, D)), full((128, 1)),
            full((D, 128)), full((D, 1)),
            full((32, D)), full((32, 1)),
            full((1, 32)), full((1, D)), full((1, 1)),
        ],
        out_specs=pl.BlockSpec((1, BT), lambda b: (0, b)),
        out_shape=jax.ShapeDtypeStruct((1, B), jnp.float32),
    )(u_T, i_T, uf_T, if_T,
      w0u, w0i, b0r, W1, b1r, W2, b2r, wamlp, wamf, bar)
    return out.reshape(B, 1)
